# baseline (speedup 1.0000x reference)
import jax
import jax.numpy as jnp
from jax.experimental import pallas as pl
from jax.experimental.pallas import tpu as pltpu
from jax.experimental.pallas import tpu_sc as plsc

_T, _D, _E, _K, _I = 2048, 1024, 8, 2, 512
_SI = 512
_BLK = 128
_NPAD = _T * _K + _E * _BLK
_NBLK = _NPAD // _BLK
_TB = 256
_W = 128
_F = 4


def _routing_kernel(x_ref, gw_ref, i0_ref, i1_ref, w0_ref, w1_ref):
    l = jnp.dot(x_ref[...], gw_ref[...], preferred_element_type=jnp.float32)
    lane = jax.lax.broadcasted_iota(jnp.int32, l.shape, 1)
    neg = jnp.float32(-1e30)
    l = jnp.where(lane < _E, l, neg)
    m0 = jnp.max(l, axis=1, keepdims=True)
    i0 = jnp.min(jnp.where(l == m0, lane, _E), axis=1, keepdims=True)
    l1 = jnp.where(lane == i0, neg, l)
    m1 = jnp.max(l1, axis=1, keepdims=True)
    i1 = jnp.min(jnp.where(l1 == m1, lane, _E), axis=1, keepdims=True)
    e1 = jnp.exp(m1 - m0)
    s = 1.0 + e1
    i0_ref[...] = jnp.broadcast_to(i0, i0_ref.shape)
    i1_ref[...] = jnp.broadcast_to(i1, i1_ref.shape)
    w0_ref[...] = jnp.broadcast_to(1.0 / s, w0_ref.shape)
    w1_ref[...] = jnp.broadcast_to(e1 / s, w1_ref.shape)


def _moe_mlp_kernel(bexp_ref, xs_ref, wgu_ref, wd_ref, y_ref):
    del bexp_ref
    gu = jnp.dot(xs_ref[...], wgu_ref[0], preferred_element_type=jnp.float32)
    g = gu[:, :_I]
    u = gu[:, _I:]
    a = g * jax.nn.sigmoid(g) * u
    y_ref[...] = jnp.dot(a, wd_ref[0], preferred_element_type=jnp.float32)


def _shared_mlp_kernel(x_ref, wgu_ref, wd_ref, o_ref):
    gu = jnp.dot(x_ref[...], wgu_ref[...], preferred_element_type=jnp.float32)
    g = gu[:, :_SI]
    u = gu[:, _SI:]
    a = g * jax.nn.sigmoid(g) * u
    o_ref[...] = jnp.dot(a, wd_ref[...], preferred_element_type=jnp.float32)


def _combine_kernel(g_ref, sh_ref, w0_ref, w1_ref, o_ref):
    o_ref[...] = (w0_ref[:, 0:1] * g_ref[0]
                  + w1_ref[:, 0:1] * g_ref[1]
                  + sh_ref[...])


def _sc_gather(data, idx, n, d):
    d2 = d // _F
    n2 = n * _F
    idx2 = (idx[:, None] * _F
            + jnp.arange(_F, dtype=idx.dtype)[None, :]).reshape(1, n2)
    data2 = data.reshape(-1, d2)
    mesh = plsc.VectorSubcoreMesh(core_axis_name="core",
                                  subcore_axis_name="subcore")

    @pl.kernel(out_type=jax.ShapeDtypeStruct((n2, d2), data.dtype), mesh=mesh)
    def k(x_hbm, i_hbm, o_hbm):
        def body(i_vmem, o_vmem):
            pltpu.sync_copy(x_hbm.at[i_vmem.at[0]], o_vmem)

        pltpu.emit_pipeline(
            body,
            grid=(n2 // _W,),
            in_specs=[pl.BlockSpec((1, _W), lambda i: (0, i))],
            out_specs=[pl.BlockSpec((_W, d2), lambda i: (i, 0))],
            core_axis_name=("core", "subcore"),
            dimension_semantics=(pltpu.PARALLEL,),
        )(i_hbm, o_hbm)

    return k(data2, idx2).reshape(n, d)


def kernel(hidden_states, gate_w, w_gate_up, w_down, sh_gate_up, sh_down):
    x = hidden_states.reshape(_T, _D)
    gwp = jnp.pad(gate_w, ((0, 0), (0, 128 - _E)))

    i0b, i1b, w0b, w1b = pl.pallas_call(
        _routing_kernel,
        grid=(_T // _TB,),
        in_specs=[pl.BlockSpec((_TB, _D), lambda i: (i, 0)),
                  pl.BlockSpec((_D, 128), lambda i: (0, 0))],
        out_specs=[pl.BlockSpec((_TB, 128), lambda i: (i, 0))] * 4,
        out_shape=[jax.ShapeDtypeStruct((_T, 128), jnp.int32),
                   jax.ShapeDtypeStruct((_T, 128), jnp.int32),
                   jax.ShapeDtypeStruct((_T, 128), jnp.float32),
                   jax.ShapeDtypeStruct((_T, 128), jnp.float32)],
    )(x, gwp)

    i0 = i0b[:, 0]
    i1 = i1b[:, 0]
    e_flat = jnp.concatenate([i0, i1])
    toks = jnp.concatenate([jnp.arange(_T, dtype=jnp.int32)] * 2)
    oh = (e_flat[:, None] == jnp.arange(_E, dtype=jnp.int32)[None, :])
    csum = jnp.cumsum(oh.astype(jnp.int32), axis=0)
    counts = csum[-1]
    rank = jnp.take_along_axis(csum, e_flat[:, None], axis=1)[:, 0] - 1
    padded = ((counts + _BLK - 1) // _BLK) * _BLK
    ends = jnp.cumsum(padded)
    offs = ends - padded
    r = offs[e_flat] + rank
    src = jnp.zeros((_NPAD,), jnp.int32).at[r].set(toks, unique_indices=True)
    bstart = jnp.arange(_NBLK, dtype=jnp.int32) * _BLK
    bexp = jnp.minimum(jnp.searchsorted(ends, bstart, side="right"),
                       _E - 1).astype(jnp.int32)

    xs = _sc_gather(x, src, _NPAD, _D)

    y = pl.pallas_call(
        _moe_mlp_kernel,
        grid_spec=pltpu.PrefetchScalarGridSpec(
            num_scalar_prefetch=1,
            grid=(_NBLK,),
            in_specs=[pl.BlockSpec((_BLK, _D), lambda i, b: (i, 0)),
                      pl.BlockSpec((1, _D, 2 * _I), lambda i, b: (b[i], 0, 0)),
                      pl.BlockSpec((1, _I, _D), lambda i, b: (b[i], 0, 0))],
            out_specs=pl.BlockSpec((_BLK, _D), lambda i, b: (i, 0)),
        ),
        out_shape=jax.ShapeDtypeStruct((_NPAD, _D), jnp.float32),
        compiler_params=pltpu.CompilerParams(
            dimension_semantics=("arbitrary",)),
    )(bexp, xs, w_gate_up, w_down)

    sh = pl.pallas_call(
        _shared_mlp_kernel,
        grid=(_T // _TB,),
        in_specs=[pl.BlockSpec((_TB, _D), lambda i: (i, 0)),
                  pl.BlockSpec((_D, 2 * _SI), lambda i: (0, 0)),
                  pl.BlockSpec((_SI, _D), lambda i: (0, 0))],
        out_specs=pl.BlockSpec((_TB, _D), lambda i: (i, 0)),
        out_shape=jax.ShapeDtypeStruct((_T, _D), jnp.float32),
    )(x, sh_gate_up, sh_down)

    g2 = _sc_gather(y, r, _K * _T, _D).reshape(_K, _T, _D)

    final = pl.pallas_call(
        _combine_kernel,
        grid=(_T // _TB,),
        in_specs=[pl.BlockSpec((_K, _TB, _D), lambda i: (0, i, 0)),
                  pl.BlockSpec((_TB, _D), lambda i: (i, 0)),
                  pl.BlockSpec((_TB, 128), lambda i: (i, 0)),
                  pl.BlockSpec((_TB, 128), lambda i: (i, 0))],
        out_specs=pl.BlockSpec((_TB, _D), lambda i: (i, 0)),
        out_shape=jax.ShapeDtypeStruct((_T, _D), jnp.float32),
    )(g2, sh, w0b, w1b)
    return final.reshape(hidden_states.shape)
